# baseline (speedup 1.0000x reference)
import functools

import jax
import jax.numpy as jnp
from jax import lax
from jax.experimental import pallas as pl
from jax.experimental.pallas import tpu as pltpu
from jax.experimental.pallas import tpu_sc as plsc

B = 16384
EMB = 16
NC = 2
NS = 16
NW = NC * NS
BPW = B // NW
CH = 128
NCH = BPW // CH
NG = BPW // 16

_mesh = plsc.VectorSubcoreMesh(core_axis_name="c", subcore_axis_name="s")


@functools.partial(
    pl.kernel,
    mesh=_mesh,
    out_type=jax.ShapeDtypeStruct((B,), jnp.float32),
    compiler_params=pltpu.CompilerParams(
        needs_layout_passes=False, use_tc_tiling_on_sc=False),
    scratch_types=[
        pltpu.VMEM((NCH, CH), jnp.int32),
        pltpu.VMEM((NCH, CH), jnp.int32),
        pltpu.VMEM((BPW, EMB), jnp.float32),
        pltpu.VMEM((BPW, EMB), jnp.float32),
        pltpu.VMEM((16, 16), jnp.float32),
        pltpu.VMEM((BPW,), jnp.float32),
        pltpu.VMEM((2, 16), jnp.float32),
        pltpu.SemaphoreType.DMA,
    ],
)
def _sc_fwd(mi_hbm, gi_hbm, m_hbm, g_hbm, wb_hbm, out_hbm,
            midx_v, gidx_v, mrows_v, grows_v, pt_v, out_v, wb_v, sem):
    wid = lax.axis_index("s") * NC + lax.axis_index("c")
    base = wid * BPW

    pltpu.sync_copy(mi_hbm.at[wid], midx_v)
    pltpu.sync_copy(gi_hbm.at[wid], gidx_v)
    pltpu.sync_copy(wb_hbm, wb_v)

    copies = []
    for j in range(NCH):
        copies.append(pltpu.async_copy(
            m_hbm.at[midx_v.at[j]], mrows_v.at[pl.ds(j * CH, CH)], sem))
        copies.append(pltpu.async_copy(
            g_hbm.at[gidx_v.at[j]], grows_v.at[pl.ds(j * CH, CH)], sem))
    for cp in copies:
        cp.wait()

    lane = lax.iota(jnp.int32, 16)
    wv = wb_v[0]
    bv = wb_v[1]
    for g in range(NG):
        acc = jnp.zeros((16,), jnp.float32)
        for j in range(16):
            r = g * 16 + j
            prod = mrows_v[r] * grows_v[r]
            s = jnp.sum(prod)
            acc = jnp.where(lane == j, s, acc)
        t = acc * wv + bv
        y = 1.0 / (1.0 + jnp.exp(-t))
        out_v[pl.ds(g * 16, 16)] = y

    pltpu.sync_copy(out_v, out_hbm.at[pl.ds(base, BPW)])


def kernel(x, m_table, g_table, W, b):
    mi = x[:, 0].reshape(NW, NCH, CH)
    gi = x[:, 1].reshape(NW, NCH, CH)
    wb = jnp.stack([jnp.full((16,), W[0, 0], jnp.float32),
                    jnp.full((16,), b[0], jnp.float32)])
    out = _sc_fwd(mi, gi, m_table, g_table, wb)
    return out.reshape(B, 1)
